# baseline (speedup 1.0000x reference)
import functools

import jax
import jax.numpy as jnp
from jax import lax
from jax.experimental import pallas as pl
from jax.experimental.pallas import tpu as pltpu
from jax.experimental.pallas import tpu_sc as plsc

D = 128
VOCAB = 178
B_TOK = 1024 * 512
NC, NS = 2, 16
NW = NC * NS
K = 64
CHUNKS = B_TOK // (NW * K)
NBUF = 8
DEPTH = NBUF // 2


def _body(idx_hbm, table_hbm, out_hbm, idx_v, rows_v, tab_v, tab_sh,
          gsem, wsem):
    sid = lax.axis_index("s")
    wid = sid * NC + lax.axis_index("c")

    @pl.when(sid == 0)
    def _():
        pltpu.sync_copy(table_hbm, tab_v)
        pltpu.sync_copy(tab_v, tab_sh)

    pltpu.sync_copy(idx_hbm.at[wid], idx_v)
    plsc.subcore_barrier()
    base = wid * (CHUNKS * K)

    def gather_copy(j, b):
        return pltpu.make_async_copy(tab_sh.at[idx_v.at[j]],
                                     rows_v.at[b], gsem)

    def write_copy(j, b):
        return pltpu.make_async_copy(rows_v.at[b],
                                     out_hbm.at[pl.ds(base + j * K, K)],
                                     wsem)

    for j in range(NBUF):
        gather_copy(j, j).start()
        if j >= DEPTH:
            gather_copy(j - DEPTH, j - DEPTH).wait()
            write_copy(j - DEPTH, j - DEPTH).start()

    def outer(i, carry):
        for b in range(NBUF):
            j = i * NBUF + b
            write_copy(j - NBUF, b).wait()
            gather_copy(j, b).start()
            pb = (b + DEPTH) % NBUF
            gather_copy(j - DEPTH, pb).wait()
            write_copy(j - DEPTH, pb).start()
        return carry

    lax.fori_loop(1, CHUNKS // NBUF, outer, 0)

    for b in range(DEPTH):
        j = CHUNKS - DEPTH + b
        gather_copy(j, j % NBUF).wait()
        write_copy(j, j % NBUF).start()
    for b in range(NBUF):
        write_copy(CHUNKS - NBUF + b, b).wait()


def kernel(input_ids, word_embeddings, text_embeddings):
    del text_embeddings
    idx = input_ids.reshape(NW, CHUNKS, K)
    run = functools.partial(
        pl.kernel,
        mesh=plsc.VectorSubcoreMesh(core_axis_name="c", subcore_axis_name="s"),
        out_type=jax.ShapeDtypeStruct((B_TOK, D), jnp.float32),
        scratch_types=[
            pltpu.VMEM((CHUNKS, K), jnp.int32),
            pltpu.VMEM((NBUF, K, D), jnp.float32),
            pltpu.VMEM((VOCAB, D), jnp.float32),
            pltpu.VMEM_SHARED((VOCAB, D), jnp.float32),
            pltpu.SemaphoreType.DMA,
            pltpu.SemaphoreType.DMA,
        ],
    )(_body)
    out = run(idx, word_embeddings)
    return out.reshape(1024, 512, D)
